# baseline (speedup 1.0000x reference)
import functools

import jax
import jax.numpy as jnp
from jax import lax
from jax.experimental import pallas as pl
from jax.experimental.pallas import tpu as pltpu
from jax.experimental.pallas import tpu_sc as plsc

B, H, W, C, K = 8, 128, 128, 256, 21
_NUM_ROWS = B * K


@functools.partial(
    pl.kernel,
    out_type=jax.ShapeDtypeStruct((_NUM_ROWS, C), jnp.float32),
    mesh=plsc.ScalarSubcoreMesh(axis_name="c", num_cores=1),
    scratch_types=[
        pltpu.SMEM((2 * _NUM_ROWS,), jnp.int32),
        pltpu.SemaphoreType.DMA,
    ],
)
def _gather_rows_scs(uv_hbm, table_hbm, out_hbm, uv_s, sem):
    pltpu.sync_copy(uv_hbm, uv_s)

    def body(i, carry):
        b = lax.div(i, K)
        h = uv_s[2 * i]
        w = uv_s[2 * i + 1]
        flat = b * (H * W) + h * W + w
        pltpu.async_copy(table_hbm.at[pl.ds(flat, 1)],
                         out_hbm.at[pl.ds(i, 1)], sem)
        return carry

    lax.fori_loop(0, _NUM_ROWS, body, 0, unroll=False)
    pltpu.make_async_copy(table_hbm.at[pl.ds(0, _NUM_ROWS)], out_hbm, sem).wait()


def kernel(inputs, uv):
    table = inputs.reshape(B * H * W, C)
    uv_flat = uv.astype(jnp.int32).reshape(_NUM_ROWS * 2)
    out = _gather_rows_scs(uv_flat, table)
    return out.reshape(B, K, C)


# speedup vs baseline: 1.1408x; 1.1408x over previous
import functools

import jax
import jax.numpy as jnp
from jax import lax
from jax.experimental import pallas as pl
from jax.experimental.pallas import tpu as pltpu
from jax.experimental.pallas import tpu_sc as plsc

B, H, W, C, K = 8, 128, 128, 256, 21

_NUM_ROWS = B * K
_R_PER_WORKER = 16
_FULL_WORKERS = _NUM_ROWS // _R_PER_WORKER

_GATHER_DNUMS = lax.GatherDimensionNumbers(
    offset_dims=(), collapsed_slice_dims=(0,), start_index_map=(0,))


def _lane_gather(x, idx):
    return lax.gather(x, idx[:, None], dimension_numbers=_GATHER_DNUMS,
                      slice_sizes=(1,),
                      mode=lax.GatherScatterMode.PROMISE_IN_BOUNDS)


def _pair_sum(uvp, lanes):
    prod = jnp.where(lax.rem(lanes, jnp.int32(2)) == 0, uvp * W, uvp)
    pair = lax.rem(2 * lanes, jnp.int32(16))
    return _lane_gather(prod, pair) + _lane_gather(prod, pair + 1)


@functools.partial(
    pl.kernel,
    out_type=jax.ShapeDtypeStruct((_NUM_ROWS, C), jnp.float32),
    mesh=plsc.VectorSubcoreMesh(core_axis_name="c", subcore_axis_name="s",
                                num_cores=1, num_subcores=11),
    scratch_types=[
        pltpu.VMEM((16,), jnp.int32),
        pltpu.VMEM((16,), jnp.int32),
        pltpu.VMEM((16,), jnp.int32),
        pltpu.VMEM((16, C), jnp.float32),
        pltpu.SemaphoreType.DMA,
    ],
)
def _gather_rows(uv_hbm, table_hbm, out_hbm, uva_v, uvb_v, idx_v, rows_v, sem):
    wid = lax.axis_index("s")

    @pl.when(wid <= _FULL_WORKERS)
    def _():
        base = wid * _R_PER_WORKER
        pltpu.sync_copy(uv_hbm.at[pl.ds(2 * base, 16)], uva_v)

        @pl.when(wid < _FULL_WORKERS)
        def _():
            pltpu.sync_copy(uv_hbm.at[pl.ds(2 * base + 16, 16)], uvb_v)

        lanes = lax.iota(jnp.int32, 16)
        hw = jnp.where(lanes < 8, _pair_sum(uva_v[...], lanes),
                       _pair_sum(uvb_v[...], lanes))
        b = lax.div(base + lanes, jnp.int32(K))
        flat = b * (H * W) + hw
        nvalid = jnp.where(wid < _FULL_WORKERS, 16, 8)
        idx_v[...] = jnp.where(lanes < nvalid, flat, 0)
        pltpu.async_copy(table_hbm.at[idx_v], rows_v, sem).wait()

        @pl.when(wid < _FULL_WORKERS)
        def _():
            pltpu.sync_copy(rows_v, out_hbm.at[pl.ds(base, _R_PER_WORKER)])

        @pl.when(wid == _FULL_WORKERS)
        def _():
            pltpu.sync_copy(rows_v.at[pl.ds(0, 8)],
                            out_hbm.at[pl.ds(_FULL_WORKERS * _R_PER_WORKER, 8)])


def kernel(inputs, uv):
    table = inputs.reshape(B * H * W, C)
    uv_flat = uv.astype(jnp.int32).reshape(_NUM_ROWS * 2)
    out = _gather_rows(uv_flat, table)
    return out.reshape(B, K, C)


# speedup vs baseline: 1.1473x; 1.0056x over previous
import functools

import jax
import jax.numpy as jnp
from jax import lax
from jax.experimental import pallas as pl
from jax.experimental.pallas import tpu as pltpu
from jax.experimental.pallas import tpu_sc as plsc

B, H, W, C, K = 8, 128, 128, 256, 21

_NUM_ROWS = B * K
_R_PER_WORKER = 16
_FULL_WORKERS = _NUM_ROWS // _R_PER_WORKER

_GATHER_DNUMS = lax.GatherDimensionNumbers(
    offset_dims=(), collapsed_slice_dims=(0,), start_index_map=(0,))


def _lane_gather(x, idx):
    return lax.gather(x, idx[:, None], dimension_numbers=_GATHER_DNUMS,
                      slice_sizes=(1,),
                      mode=lax.GatherScatterMode.PROMISE_IN_BOUNDS)


def _pair_sum(uvp, lanes):
    prod = jnp.where(lax.rem(lanes, jnp.int32(2)) == 0, uvp * W, uvp)
    pair = lax.rem(2 * lanes, jnp.int32(16))
    return _lane_gather(prod, pair) + _lane_gather(prod, pair + 1)


@functools.partial(
    pl.kernel,
    out_type=jax.ShapeDtypeStruct((_NUM_ROWS, C), jnp.float32),
    mesh=plsc.VectorSubcoreMesh(core_axis_name="c", subcore_axis_name="s",
                                num_cores=1, num_subcores=11),
    compiler_params=pltpu.CompilerParams(skip_device_barrier=True),
    scratch_types=[
        pltpu.VMEM((16,), jnp.int32),
        pltpu.VMEM((16,), jnp.int32),
        pltpu.VMEM((16,), jnp.int32),
        pltpu.VMEM((16, C), jnp.float32),
        pltpu.SemaphoreType.DMA,
    ],
)
def _gather_rows(uv_hbm, table_hbm, out_hbm, uva_v, uvb_v, idx_v, rows_v, sem):
    wid = lax.axis_index("s")

    @pl.when(wid <= _FULL_WORKERS)
    def _():
        base = wid * _R_PER_WORKER
        pltpu.sync_copy(uv_hbm.at[pl.ds(2 * base, 16)], uva_v)

        @pl.when(wid < _FULL_WORKERS)
        def _():
            pltpu.sync_copy(uv_hbm.at[pl.ds(2 * base + 16, 16)], uvb_v)

        lanes = lax.iota(jnp.int32, 16)
        hw = jnp.where(lanes < 8, _pair_sum(uva_v[...], lanes),
                       _pair_sum(uvb_v[...], lanes))
        b = lax.div(base + lanes, jnp.int32(K))
        flat = b * (H * W) + hw
        nvalid = jnp.where(wid < _FULL_WORKERS, 16, 8)
        idx_v[...] = jnp.where(lanes < nvalid, flat, 0)
        pltpu.async_copy(table_hbm.at[idx_v], rows_v, sem).wait()

        @pl.when(wid < _FULL_WORKERS)
        def _():
            pltpu.sync_copy(rows_v, out_hbm.at[pl.ds(base, _R_PER_WORKER)])

        @pl.when(wid == _FULL_WORKERS)
        def _():
            pltpu.sync_copy(rows_v.at[pl.ds(0, 8)],
                            out_hbm.at[pl.ds(_FULL_WORKERS * _R_PER_WORKER, 8)])


def kernel(inputs, uv):
    table = inputs.reshape(B * H * W, C)
    uv_flat = uv.astype(jnp.int32).reshape(_NUM_ROWS * 2)
    out = _gather_rows(uv_flat, table)
    return out.reshape(B, K, C)


# speedup vs baseline: 1.1693x; 1.0192x over previous
import functools

import jax
import jax.numpy as jnp
from jax import lax
from jax.experimental import pallas as pl
from jax.experimental.pallas import tpu as pltpu
from jax.experimental.pallas import tpu_sc as plsc

B, H, W, C, K = 8, 128, 128, 256, 21

_NUM_ROWS = B * K
_R_PER_WORKER = 16
_NW = 11
_LAST_BASE = _NUM_ROWS - _R_PER_WORKER

_GATHER_DNUMS = lax.GatherDimensionNumbers(
    offset_dims=(), collapsed_slice_dims=(0,), start_index_map=(0,))


def _lane_gather(x, idx):
    return lax.gather(x, idx[:, None], dimension_numbers=_GATHER_DNUMS,
                      slice_sizes=(1,),
                      mode=lax.GatherScatterMode.PROMISE_IN_BOUNDS)


def _pair_sum(uvp, lanes):
    prod = jnp.where(lax.rem(lanes, jnp.int32(2)) == 0, uvp * W, uvp)
    pair = lax.rem(2 * lanes, jnp.int32(16))
    return _lane_gather(prod, pair) + _lane_gather(prod, pair + 1)


@functools.partial(
    pl.kernel,
    out_type=jax.ShapeDtypeStruct((_NUM_ROWS, C), jnp.float32),
    mesh=plsc.VectorSubcoreMesh(core_axis_name="c", subcore_axis_name="s",
                                num_cores=1, num_subcores=_NW),
    scratch_types=[
        pltpu.VMEM((2 * _R_PER_WORKER,), jnp.int32),
        pltpu.VMEM((_R_PER_WORKER,), jnp.int32),
        pltpu.VMEM((_R_PER_WORKER, C), jnp.float32),
        pltpu.SemaphoreType.DMA,
    ],
)
def _gather_rows(uv_hbm, table_hbm, out_hbm, uvw_v, idx_v, rows_v, sem):
    wid = lax.axis_index("s")
    base = jnp.minimum(wid * _R_PER_WORKER, _LAST_BASE)
    pltpu.sync_copy(uv_hbm.at[pl.ds(pl.multiple_of(2 * base, 16), 32)], uvw_v)
    lanes = lax.iota(jnp.int32, 16)
    hw = jnp.where(lanes < 8,
                   _pair_sum(uvw_v[pl.ds(0, 16)], lanes),
                   _pair_sum(uvw_v[pl.ds(16, 16)], lanes))
    b = lax.div(base + lanes, jnp.int32(K))
    idx_v[...] = b * (H * W) + hw
    pltpu.async_copy(table_hbm.at[idx_v], rows_v, sem).wait()

    @pl.when(wid < _NW - 1)
    def _():
        pltpu.sync_copy(rows_v, out_hbm.at[pl.ds(base, _R_PER_WORKER)])

    @pl.when(wid == _NW - 1)
    def _():
        pltpu.sync_copy(rows_v.at[pl.ds(8, 8)],
                        out_hbm.at[pl.ds(_LAST_BASE + 8, 8)])


def kernel(inputs, uv):
    table = inputs.reshape(B * H * W, C)
    uv_flat = uv.astype(jnp.int32).reshape(_NUM_ROWS * 2)
    out = _gather_rows(uv_flat, table)
    return out.reshape(B, K, C)
